# Initial kernel scaffold; baseline (speedup 1.0000x reference)
import jax
import jax.numpy as jnp
from jax.experimental import pallas as pl


def kernel(x, edge_index, edge_weight, Wc1, bc1, Wc2, bc2, g1, be1, g2, be2, Wih1, Whh1, bih1, bhh1, Wih2, Whh2, bih2, bhh2, Wl1, bl1, Wl2, bl2):
    raise NotImplementedError("write your pallas kernel here")


# speedup vs baseline: 1.1745x; 1.1745x over previous
import functools

import jax
import jax.numpy as jnp
from jax.experimental import pallas as pl
from jax.experimental.pallas import tpu as pltpu

IN_CH = 128
HID = 128
NUM_NODES = 10000
WINDOW = 4
N_TOT = NUM_NODES * WINDOW
BN_SCALE = 1.0 / (1.0 + 1e-5) ** 0.5


def _lstm_head_body(xc_ref, s_ref, wih1_ref, whh1_ref, b1_ref,
                    wih2_ref, whh2_ref, b2_ref, wl1a_ref, wl1b_ref,
                    wl1c_ref, bl1_ref, wl2_ref, bl2_ref, out_ref):
    B = xc_ref.shape[1]
    h1 = jnp.zeros((B, HID), jnp.float32)
    c1 = jnp.zeros((B, HID), jnp.float32)
    h2 = jnp.zeros((B, HID), jnp.float32)
    c2 = jnp.zeros((B, HID), jnp.float32)
    wih1 = wih1_ref[...]
    whh1 = whh1_ref[...]
    wih2 = wih2_ref[...]
    whh2 = whh2_ref[...]
    b1 = b1_ref[...]
    b2 = b2_ref[...]

    def lstm_step(g, c):
        i = jax.nn.sigmoid(g[:, 0:HID])
        f = jax.nn.sigmoid(g[:, HID:2 * HID])
        gg = jnp.tanh(g[:, 2 * HID:3 * HID])
        o = jax.nn.sigmoid(g[:, 3 * HID:4 * HID])
        c = f * c + i * gg
        h = o * jnp.tanh(c)
        return h, c

    for t in range(WINDOW):
        xt = xc_ref[t]
        g1 = jnp.dot(xt, wih1, preferred_element_type=jnp.float32)
        g1 = g1 + jnp.dot(h1, whh1, preferred_element_type=jnp.float32) + b1
        h1, c1 = lstm_step(g1, c1)
        g2 = jnp.dot(h1, wih2, preferred_element_type=jnp.float32)
        g2 = g2 + jnp.dot(h2, whh2, preferred_element_type=jnp.float32) + b2
        h2, c2 = lstm_step(g2, c2)

    ht = jnp.dot(jax.nn.relu(h1), wl1a_ref[...], preferred_element_type=jnp.float32)
    ht = ht + jnp.dot(jax.nn.relu(h2), wl1b_ref[...], preferred_element_type=jnp.float32)
    ht = ht + jnp.dot(jax.nn.relu(s_ref[...]), wl1c_ref[...], preferred_element_type=jnp.float32)
    ht = jax.nn.relu(ht + bl1_ref[...])
    out_ref[...] = jnp.dot(ht, wl2_ref[...], preferred_element_type=jnp.float32) + bl2_ref[...]


def _lstm_head(xc, s, wih1t, whh1t, b1, wih2t, whh2t, b2,
               wl1a, wl1b, wl1c, bl1, wl2t, bl2):
    B = 1000
    grid = (NUM_NODES // B,)
    full = lambda shape: pl.BlockSpec(shape, lambda i: (0,) * len(shape))
    return pl.pallas_call(
        _lstm_head_body,
        grid=grid,
        in_specs=[
            pl.BlockSpec((WINDOW, B, 2 * HID), lambda i: (0, i, 0)),
            pl.BlockSpec((B, s.shape[1]), lambda i: (i, 0)),
            full(wih1t.shape), full(whh1t.shape), full(b1.shape),
            full(wih2t.shape), full(whh2t.shape), full(b2.shape),
            full(wl1a.shape), full(wl1b.shape), full(wl1c.shape),
            full(bl1.shape), full(wl2t.shape), full(bl2.shape),
        ],
        out_specs=pl.BlockSpec((B, 7), lambda i: (i, 0)),
        out_shape=jax.ShapeDtypeStruct((NUM_NODES, 7), jnp.float32),
    )(xc, s, wih1t, whh1t, b1, wih2t, whh2t, b2, wl1a, wl1b, wl1c, bl1, wl2t, bl2)


def _gcn_layer(x, row, col, ew, norm, dinv2, W, b, g, be):
    h = x @ W
    agg = jnp.zeros_like(h).at[col].add(norm[:, None] * h[row])
    agg = agg + dinv2[:, None] * h
    return g * (jax.nn.relu(agg + b) * BN_SCALE) + be


def kernel(x, edge_index, edge_weight, Wc1, bc1, Wc2, bc2, g1, be1, g2, be2,
           Wih1, Whh1, bih1, bhh1, Wih2, Whh2, bih2, bhh2, Wl1, bl1, Wl2, bl2):
    row = edge_index[0]
    col = edge_index[1]
    deg = jnp.ones((N_TOT,), jnp.float32).at[col].add(edge_weight)
    dinv = jnp.where(deg > 0, jax.lax.rsqrt(deg), 0.0)
    norm = dinv[row] * edge_weight * dinv[col]
    dinv2 = dinv * dinv

    h1 = _gcn_layer(x, row, col, edge_weight, norm, dinv2, Wc1, bc1, g1, be1)
    h2 = _gcn_layer(h1, row, col, edge_weight, norm, dinv2, Wc2, bc2, g2, be2)

    s = jnp.concatenate(
        [x[0:NUM_NODES]]
        + [x[t * NUM_NODES:(t + 1) * NUM_NODES, IN_CH - 1:IN_CH] for t in range(1, WINDOW)],
        axis=1)

    xc = jnp.concatenate([h1, h2], axis=1).reshape(WINDOW, NUM_NODES, 2 * HID)

    b1 = (bih1 + bhh1).reshape(1, 4 * HID)
    b2 = (bih2 + bhh2).reshape(1, 4 * HID)
    wl1t = Wl1.T
    return _lstm_head(xc, s, Wih1.T, Whh1.T, b1, Wih2.T, Whh2.T, b2,
                      wl1t[0:HID], wl1t[HID:2 * HID], wl1t[2 * HID:],
                      bl1.reshape(1, HID), Wl2.T, bl2.reshape(1, 7))


# speedup vs baseline: 8.3893x; 7.1429x over previous
import functools

import jax
import jax.numpy as jnp
from jax import lax
from jax.experimental import pallas as pl
from jax.experimental.pallas import tpu as pltpu
from jax.experimental.pallas import tpu_sc as plsc

IN_CH = 128
HID = 128
NUM_NODES = 10000
WINDOW = 4
N_TOT = NUM_NODES * WINDOW
N_EDGES = 500000
E_PAD = 524288
EROWS = E_PAD // 128
BN_SCALE = 1.0 / (1.0 + 1e-5) ** 0.5
NC, NS = 2, 16
ROWS_PER_TILE = N_TOT // NS
DEG_CHUNK = 2504
DEG_LAST = N_TOT - 15 * DEG_CHUNK


def _sc_mesh():
    return plsc.VectorSubcoreMesh(core_axis_name="c", subcore_axis_name="s",
                                  num_cores=NC, num_subcores=NS)


def _sc_deg_body(col2d, ew2d, p0, p1, col_v, ew_v, zbuf, acc_sh):
    cid = lax.axis_index("c")
    tid = lax.axis_index("s")
    outs = (p0, p1)

    def zero(i, c):
        zbuf[pl.ds(i * 16, 16)] = jnp.zeros((16,), jnp.float32)
        return c
    lax.fori_loop(0, DEG_CHUNK // 16, zero, 0, unroll=8)
    off = tid * DEG_CHUNK

    @pl.when(tid < NS - 1)
    def _():
        pltpu.sync_copy(zbuf, acc_sh.at[pl.ds(off, DEG_CHUNK)])

    @pl.when(tid == NS - 1)
    def _():
        pltpu.sync_copy(zbuf.at[pl.ds(0, DEG_LAST)], acc_sh.at[pl.ds(off, DEG_LAST)])

    plsc.subcore_barrier()
    blk0 = cid * (EROWS // 2) + tid * (EROWS // 2 // NS)

    def blk(b, c):
        base = blk0 + b * 8
        pltpu.sync_copy(col2d.at[pl.ds(base, 8)], col_v)
        pltpu.sync_copy(ew2d.at[pl.ds(base, 8)], ew_v)
        for k in range(8):
            pltpu.sync_copy(ew_v.at[k], acc_sh.at[col_v.at[k]], add=True)
        return c
    lax.fori_loop(0, EROWS // 2 // NS // 8, blk, 0)
    plsc.subcore_barrier()

    for c in range(NC):
        @pl.when(cid == c)
        def _(c=c):
            @pl.when(tid < NS - 1)
            def _():
                pltpu.sync_copy(acc_sh.at[pl.ds(off, DEG_CHUNK)], zbuf)
                pltpu.sync_copy(zbuf, outs[c].at[pl.ds(off, DEG_CHUNK)])

            @pl.when(tid == NS - 1)
            def _():
                pltpu.sync_copy(acc_sh.at[pl.ds(off, DEG_LAST)],
                                zbuf.at[pl.ds(0, DEG_LAST)])
                pltpu.sync_copy(zbuf.at[pl.ds(0, DEG_LAST)],
                                outs[c].at[pl.ds(off, DEG_LAST)])


def _sc_deg(col2d, ew2d):
    f = pl.kernel(
        _sc_deg_body,
        out_type=[jax.ShapeDtypeStruct((N_TOT,), jnp.float32),
                  jax.ShapeDtypeStruct((N_TOT,), jnp.float32)],
        mesh=_sc_mesh(),
        scratch_types=[
            pltpu.VMEM((8, 128), jnp.int32),
            pltpu.VMEM((8, 128), jnp.float32),
            pltpu.VMEM((DEG_CHUNK,), jnp.float32),
            pltpu.VMEM_SHARED((N_TOT,), jnp.float32),
        ],
    )
    return f(col2d, ew2d)


def _sc_agg_body(hs0, hs1, hs2, hs3, row2d, col2d, ew2d,
                 o0, o1, o2, o3,
                 row_v, col_v, ew_v, buf0, buf1, acc_sh, sem0, sem1):
    cid = lax.axis_index("c")
    tid = lax.axis_index("s")
    tabs = (hs0, hs1, hs2, hs3)
    outs = (o0, o1, o2, o3)
    r0 = tid * DEG_CHUNK
    rows_per_tile_blk = EROWS // NS

    def _ranged_copy(mk):
        @pl.when(tid < NS - 1)
        def _():
            mk(DEG_CHUNK)

        @pl.when(tid == NS - 1)
        def _():
            mk(DEG_LAST)

    for q in range(4):
        @pl.when(cid == q // 2)
        def _(q=q):
            tab = tabs[q]
            out = outs[q]
            _ranged_copy(lambda n: pltpu.sync_copy(
                tab.at[pl.ds(r0, n)], acc_sh.at[pl.ds(r0, n)]))
            plsc.subcore_barrier()
            blk0 = tid * rows_per_tile_blk

            def blk(b, c):
                base = blk0 + b * 8
                pltpu.sync_copy(row2d.at[pl.ds(base, 8)], row_v)
                pltpu.sync_copy(col2d.at[pl.ds(base, 8)], col_v)
                pltpu.sync_copy(ew2d.at[pl.ds(base, 8)], ew_v)
                pltpu.async_copy(tab.at[row_v.at[0]], buf0, sem0)
                for k in range(8):
                    cur, csem = (buf0, sem0) if k % 2 == 0 else (buf1, sem1)
                    nxt, nsem = (buf1, sem1) if k % 2 == 0 else (buf0, sem0)
                    pltpu.make_async_copy(tab.at[row_v.at[k]], cur, csem).wait()
                    if k < 7:
                        pltpu.async_copy(tab.at[row_v.at[k + 1]], nxt, nsem)

                    def scale(i, cc, k=k, cur=cur):
                        ew16 = ew_v[k, pl.ds(i * 16, 16)]
                        for j in range(16):
                            s = ew16[j]
                            e = i * 16 + j
                            cur[e, 0:16] = cur[e, 0:16] * s
                            cur[e, 16:32] = cur[e, 16:32] * s
                        return cc
                    lax.fori_loop(0, 8, scale, 0)
                    pltpu.sync_copy(cur, acc_sh.at[col_v.at[k]], add=True)
                return c
            lax.fori_loop(0, rows_per_tile_blk // 8, blk, 0)
            plsc.subcore_barrier()
            _ranged_copy(lambda n: pltpu.sync_copy(
                acc_sh.at[pl.ds(r0, n)], out.at[pl.ds(r0, n)]))
            plsc.subcore_barrier()


def _sc_agg(hsq, row2d, col2d, ew2d):
    f = pl.kernel(
        _sc_agg_body,
        out_type=[jax.ShapeDtypeStruct((N_TOT, 32), jnp.float32)] * 4,
        mesh=_sc_mesh(),
        scratch_types=[
            pltpu.VMEM((8, 128), jnp.int32),
            pltpu.VMEM((8, 128), jnp.int32),
            pltpu.VMEM((8, 128), jnp.float32),
            pltpu.VMEM((128, 32), jnp.float32),
            pltpu.VMEM((128, 32), jnp.float32),
            pltpu.VMEM_SHARED((N_TOT, 32), jnp.float32),
            pltpu.SemaphoreType.DMA,
            pltpu.SemaphoreType.DMA,
        ],
        compiler_params=pltpu.CompilerParams(use_tc_tiling_on_sc=False),
    )
    return f(hsq[0], hsq[1], hsq[2], hsq[3], row2d, col2d, ew2d)


_R = 1000


def _k1_body(x_ref, w_ref, p0_ref, p1_ref, q0, q1, q2, q3):
    dinv = lax.rsqrt(1.0 + p0_ref[...] + p1_ref[...])
    hs = dinv * jnp.dot(x_ref[...], w_ref[...], preferred_element_type=jnp.float32)
    for i, o in enumerate((q0, q1, q2, q3)):
        o[...] = hs[:, 32 * i:32 * i + 32]


def _k1(x, w, p0c, p1c):
    full = lambda a: pl.BlockSpec(a.shape, lambda i: (0,) * a.ndim)
    return pl.pallas_call(
        _k1_body,
        grid=(N_TOT // _R,),
        in_specs=[pl.BlockSpec((_R, IN_CH), lambda i: (i, 0)), full(w),
                  pl.BlockSpec((_R, 1), lambda i: (i, 0)),
                  pl.BlockSpec((_R, 1), lambda i: (i, 0))],
        out_specs=[pl.BlockSpec((_R, 32), lambda i: (i, 0))] * 4,
        out_shape=[jax.ShapeDtypeStruct((N_TOT, 32), jnp.float32)] * 4,
    )(x, w, p0c, p1c)


def _k2_body(a0, a1, a2, a3, p0_ref, p1_ref, w_ref, b_ref, g_ref, be_ref,
             h1_ref, q0, q1, q2, q3):
    dinv = lax.rsqrt(1.0 + p0_ref[...] + p1_ref[...])
    agg = jnp.concatenate([a0[...], a1[...], a2[...], a3[...]], axis=1)
    h1 = g_ref[...] * (jax.nn.relu(dinv * agg + b_ref[...]) * BN_SCALE) + be_ref[...]
    h1_ref[...] = h1
    hs2 = dinv * jnp.dot(h1, w_ref[...], preferred_element_type=jnp.float32)
    for i, o in enumerate((q0, q1, q2, q3)):
        o[...] = hs2[:, 32 * i:32 * i + 32]


def _k2(aggq, p0c, p1c, w, b, g, be):
    full = lambda a: pl.BlockSpec(a.shape, lambda i: (0,) * a.ndim)
    return pl.pallas_call(
        _k2_body,
        grid=(N_TOT // _R,),
        in_specs=[pl.BlockSpec((_R, 32), lambda i: (i, 0))] * 4
        + [pl.BlockSpec((_R, 1), lambda i: (i, 0)),
           pl.BlockSpec((_R, 1), lambda i: (i, 0)),
           full(w), full(b), full(g), full(be)],
        out_specs=[pl.BlockSpec((_R, IN_CH), lambda i: (i, 0))]
        + [pl.BlockSpec((_R, 32), lambda i: (i, 0))] * 4,
        out_shape=[jax.ShapeDtypeStruct((N_TOT, IN_CH), jnp.float32)]
        + [jax.ShapeDtypeStruct((N_TOT, 32), jnp.float32)] * 4,
    )(aggq[0], aggq[1], aggq[2], aggq[3], p0c, p1c, w, b, g, be)


_B3 = 1000


def _k3_body(h1_ref, a0, a1, a2, a3, p0_ref, p1_ref, s_ref,
             bc2_ref, g2_ref, be2_ref,
             wih1_ref, whh1_ref, b1_ref, wih2_ref, whh2_ref, b2_ref,
             wl1a_ref, wl1b_ref, wl1c_ref, bl1_ref, wl2_ref, bl2_ref, out_ref):
    B = _B3
    h1s = jnp.zeros((B, HID), jnp.float32)
    c1s = jnp.zeros((B, HID), jnp.float32)
    h2s = jnp.zeros((B, HID), jnp.float32)
    c2s = jnp.zeros((B, HID), jnp.float32)
    wih1 = wih1_ref[...]
    whh1 = whh1_ref[...]
    wih2 = wih2_ref[...]
    whh2 = whh2_ref[...]
    b1 = b1_ref[...]
    b2 = b2_ref[...]

    def lstm_step(g, c):
        i = jax.nn.sigmoid(g[:, 0:HID])
        f = jax.nn.sigmoid(g[:, HID:2 * HID])
        gg = jnp.tanh(g[:, 2 * HID:3 * HID])
        o = jax.nn.sigmoid(g[:, 3 * HID:4 * HID])
        c = f * c + i * gg
        return o * jnp.tanh(c), c

    for t in range(WINDOW):
        dinv = lax.rsqrt(1.0 + p0_ref[t] + p1_ref[t])
        agg = jnp.concatenate([a0[t], a1[t], a2[t], a3[t]], axis=1)
        h2t = g2_ref[...] * (jax.nn.relu(dinv * agg + bc2_ref[...]) * BN_SCALE) + be2_ref[...]
        xt1 = h1_ref[t]
        gates1 = (jnp.dot(xt1, wih1[0:HID], preferred_element_type=jnp.float32)
                  + jnp.dot(h2t, wih1[HID:2 * HID], preferred_element_type=jnp.float32)
                  + jnp.dot(h1s, whh1, preferred_element_type=jnp.float32) + b1)
        h1s, c1s = lstm_step(gates1, c1s)
        gates2 = (jnp.dot(h1s, wih2, preferred_element_type=jnp.float32)
                  + jnp.dot(h2s, whh2, preferred_element_type=jnp.float32) + b2)
        h2s, c2s = lstm_step(gates2, c2s)

    ht = jnp.dot(jax.nn.relu(h1s), wl1a_ref[...], preferred_element_type=jnp.float32)
    ht = ht + jnp.dot(jax.nn.relu(h2s), wl1b_ref[...], preferred_element_type=jnp.float32)
    ht = ht + jnp.dot(jax.nn.relu(s_ref[...]), wl1c_ref[...], preferred_element_type=jnp.float32)
    ht = jax.nn.relu(ht + bl1_ref[...])
    out_ref[...] = jnp.dot(ht, wl2_ref[...], preferred_element_type=jnp.float32) + bl2_ref[...]


def _k3(h1r, aggr, p0r, p1r, s, bc2, g2, be2,
        wih1t, whh1t, b1, wih2t, whh2t, b2, wl1a, wl1b, wl1c, bl1, wl2t, bl2):
    full = lambda a: pl.BlockSpec(a.shape, lambda i: (0,) * a.ndim)
    B = _B3
    return pl.pallas_call(
        _k3_body,
        grid=(NUM_NODES // B,),
        in_specs=[pl.BlockSpec((WINDOW, B, IN_CH), lambda i: (0, i, 0))]
        + [pl.BlockSpec((WINDOW, B, 32), lambda i: (0, i, 0))] * 4
        + [pl.BlockSpec((WINDOW, B, 1), lambda i: (0, i, 0))] * 2
        + [pl.BlockSpec((B, s.shape[1]), lambda i: (i, 0))]
        + [full(a) for a in (bc2, g2, be2, wih1t, whh1t, b1, wih2t, whh2t, b2,
                             wl1a, wl1b, wl1c, bl1, wl2t, bl2)],
        out_specs=pl.BlockSpec((B, 7), lambda i: (i, 0)),
        out_shape=jax.ShapeDtypeStruct((NUM_NODES, 7), jnp.float32),
    )(h1r, aggr[0], aggr[1], aggr[2], aggr[3], p0r, p1r, s,
      bc2, g2, be2, wih1t, whh1t, b1, wih2t, whh2t, b2,
      wl1a, wl1b, wl1c, bl1, wl2t, bl2)


def kernel(x, edge_index, edge_weight, Wc1, bc1, Wc2, bc2, g1, be1, g2, be2,
           Wih1, Whh1, bih1, bhh1, Wih2, Whh2, bih2, bhh2, Wl1, bl1, Wl2, bl2):
    pad = E_PAD - N_EDGES
    pidx = (jnp.arange(pad, dtype=jnp.int32) * 79) % N_TOT
    row2d = jnp.concatenate([edge_index[0], pidx]).reshape(EROWS, 128)
    col2d = jnp.concatenate([edge_index[1], pidx]).reshape(EROWS, 128)
    ew2d = jnp.concatenate([edge_weight, jnp.zeros((pad,), jnp.float32)]).reshape(EROWS, 128)

    p0, p1 = _sc_deg(col2d, ew2d)
    p0c = p0.reshape(N_TOT, 1)
    p1c = p1.reshape(N_TOT, 1)

    hs1q = _k1(x, Wc1, p0c, p1c)
    agg1q = _sc_agg(hs1q, row2d, col2d, ew2d)
    h1, *hs2q = _k2(agg1q, p0c, p1c, Wc2, bc1.reshape(1, HID),
                    g1.reshape(1, HID), be1.reshape(1, HID))
    agg2q = _sc_agg(hs2q, row2d, col2d, ew2d)

    s = jnp.concatenate(
        [x[0:NUM_NODES]]
        + [x[t * NUM_NODES:(t + 1) * NUM_NODES, IN_CH - 1:IN_CH] for t in range(1, WINDOW)],
        axis=1)

    wl1t = Wl1.T
    return _k3(
        h1.reshape(WINDOW, NUM_NODES, IN_CH),
        [a.reshape(WINDOW, NUM_NODES, 32) for a in agg2q],
        p0.reshape(WINDOW, NUM_NODES, 1), p1.reshape(WINDOW, NUM_NODES, 1),
        s, bc2.reshape(1, HID), g2.reshape(1, HID), be2.reshape(1, HID),
        Wih1.T, Whh1.T, (bih1 + bhh1).reshape(1, 4 * HID),
        Wih2.T, Whh2.T, (bih2 + bhh2).reshape(1, 4 * HID),
        wl1t[0:HID], wl1t[HID:2 * HID], wl1t[2 * HID:],
        bl1.reshape(1, HID), Wl2.T, bl2.reshape(1, 7))


# speedup vs baseline: 13.4932x; 1.6084x over previous
import functools

import jax
import jax.numpy as jnp
from jax import lax
from jax.experimental import pallas as pl
from jax.experimental.pallas import tpu as pltpu
from jax.experimental.pallas import tpu_sc as plsc

IN_CH = 128
HID = 128
NUM_NODES = 10000
WINDOW = 4
N_TOT = NUM_NODES * WINDOW
N_EDGES = 500000
E_PAD = 524288
EROWS = E_PAD // 128
BN_SCALE = 1.0 / (1.0 + 1e-5) ** 0.5
NC, NS = 2, 16
ROWS_PER_TILE = N_TOT // NS
DEG_CHUNK = 2504
DEG_LAST = N_TOT - 15 * DEG_CHUNK


def _sc_mesh():
    return plsc.VectorSubcoreMesh(core_axis_name="c", subcore_axis_name="s",
                                  num_cores=NC, num_subcores=NS)


def _sc_deg_body(col2d, ew2d, p0, p1, col_v, ew_v, zbuf, acc_sh):
    cid = lax.axis_index("c")
    tid = lax.axis_index("s")
    outs = (p0, p1)

    def zero(i, c):
        zbuf[pl.ds(i * 16, 16)] = jnp.zeros((16,), jnp.float32)
        return c
    lax.fori_loop(0, DEG_CHUNK // 16, zero, 0, unroll=8)
    off = tid * DEG_CHUNK

    @pl.when(tid < NS - 1)
    def _():
        pltpu.sync_copy(zbuf, acc_sh.at[pl.ds(off, DEG_CHUNK)])

    @pl.when(tid == NS - 1)
    def _():
        pltpu.sync_copy(zbuf.at[pl.ds(0, DEG_LAST)], acc_sh.at[pl.ds(off, DEG_LAST)])

    plsc.subcore_barrier()
    blk0 = cid * (EROWS // 2) + tid * (EROWS // 2 // NS)

    def blk(b, c):
        base = blk0 + b * 8
        pltpu.sync_copy(col2d.at[pl.ds(base, 8)], col_v)
        pltpu.sync_copy(ew2d.at[pl.ds(base, 8)], ew_v)
        for k in range(8):
            pltpu.sync_copy(ew_v.at[k], acc_sh.at[col_v.at[k]], add=True)
        return c
    lax.fori_loop(0, EROWS // 2 // NS // 8, blk, 0)
    plsc.subcore_barrier()

    for c in range(NC):
        @pl.when(cid == c)
        def _(c=c):
            @pl.when(tid < NS - 1)
            def _():
                pltpu.sync_copy(acc_sh.at[pl.ds(off, DEG_CHUNK)], zbuf)
                pltpu.sync_copy(zbuf, outs[c].at[pl.ds(off, DEG_CHUNK)])

            @pl.when(tid == NS - 1)
            def _():
                pltpu.sync_copy(acc_sh.at[pl.ds(off, DEG_LAST)],
                                zbuf.at[pl.ds(0, DEG_LAST)])
                pltpu.sync_copy(zbuf.at[pl.ds(0, DEG_LAST)],
                                outs[c].at[pl.ds(off, DEG_LAST)])


def _sc_deg(col2d, ew2d):
    f = pl.kernel(
        _sc_deg_body,
        out_type=[jax.ShapeDtypeStruct((N_TOT,), jnp.float32),
                  jax.ShapeDtypeStruct((N_TOT,), jnp.float32)],
        mesh=_sc_mesh(),
        scratch_types=[
            pltpu.VMEM((8, 128), jnp.int32),
            pltpu.VMEM((8, 128), jnp.float32),
            pltpu.VMEM((DEG_CHUNK,), jnp.float32),
            pltpu.VMEM_SHARED((N_TOT,), jnp.float32),
        ],
    )
    return f(col2d, ew2d)


def _sc_agg_body(hs0, hs1, hs2, hs3, row2d, col2d, ew2d,
                 o0, o1, o2, o3,
                 row_a, col_a, ew_a, buf0, buf1, buf2, buf3, acc_sh,
                 g0, g1, g2, g3, s0, s1, s2, s3):
    cid = lax.axis_index("c")
    tid = lax.axis_index("s")
    tabs = (hs0, hs1, hs2, hs3)
    outs = (o0, o1, o2, o3)
    bufs = (buf0, buf1, buf2, buf3)
    gsem = (g0, g1, g2, g3)
    ssem = (s0, s1, s2, s3)
    r0 = tid * DEG_CHUNK
    nrows = EROWS // NS

    def _ranged_copy(mk):
        @pl.when(tid < NS - 1)
        def _():
            mk(DEG_CHUNK)

        @pl.when(tid == NS - 1)
        def _():
            mk(DEG_LAST)

    blk0 = tid * nrows
    SB = 64

    for q in range(4):
        @pl.when(cid == q // 2)
        def _(q=q):
            tab = tabs[q]
            out = outs[q]
            _ranged_copy(lambda n: pltpu.sync_copy(
                tab.at[pl.ds(r0, n)], acc_sh.at[pl.ds(r0, n)]))
            plsc.subcore_barrier()

            def sblk(sb, c):
                pltpu.sync_copy(row2d.at[pl.ds(blk0 + sb * SB, SB)], row_a)
                pltpu.sync_copy(col2d.at[pl.ds(blk0 + sb * SB, SB)], col_a)
                pltpu.sync_copy(ew2d.at[pl.ds(blk0 + sb * SB, SB)], ew_a)

                pltpu.async_copy(tab.at[row_a.at[0]], bufs[0], gsem[0])
                pltpu.async_copy(tab.at[row_a.at[1]], bufs[1], gsem[1])

                def quad(k, cc):
                    for j in range(4):
                        s = 4 * k + j
                        jn = (j + 2) % 4

                        @pl.when(s + 2 < SB)
                        def _(s=s, j=j, jn=jn):
                            @pl.when(s >= 2)
                            def _():
                                pltpu.make_async_copy(
                                    bufs[jn], acc_sh.at[col_a.at[0]], ssem[jn]).wait()
                            pltpu.async_copy(tab.at[row_a.at[s + 2]], bufs[jn], gsem[jn])

                        pltpu.make_async_copy(tab.at[row_a.at[0]], bufs[j], gsem[j]).wait()

                        def scale(i, ci, s=s, j=j):
                            ew16 = ew_a[s, pl.ds(i * 16, 16)]
                            for u in range(16):
                                w = ew16[u]
                                e = i * 16 + u
                                bufs[j][e, 0:16] = bufs[j][e, 0:16] * w
                                bufs[j][e, 16:32] = bufs[j][e, 16:32] * w
                            return ci
                        lax.fori_loop(0, 8, scale, 0)
                        pltpu.async_copy(bufs[j], acc_sh.at[col_a.at[s]], ssem[j], add=True)
                    return cc
                lax.fori_loop(0, SB // 4, quad, 0)
                for j in range(4):
                    pltpu.make_async_copy(bufs[j], acc_sh.at[col_a.at[0]], ssem[j]).wait()
                return c
            lax.fori_loop(0, nrows // SB, sblk, 0)
            plsc.subcore_barrier()
            _ranged_copy(lambda n: pltpu.sync_copy(
                acc_sh.at[pl.ds(r0, n)], out.at[pl.ds(r0, n)]))
            plsc.subcore_barrier()


def _sc_agg(hsq, row2d, col2d, ew2d):
    f = pl.kernel(
        _sc_agg_body,
        out_type=[jax.ShapeDtypeStruct((N_TOT, 32), jnp.float32)] * 4,
        mesh=_sc_mesh(),
        scratch_types=[
            pltpu.VMEM((64, 128), jnp.int32),
            pltpu.VMEM((64, 128), jnp.int32),
            pltpu.VMEM((64, 128), jnp.float32),
            pltpu.VMEM((128, 32), jnp.float32),
            pltpu.VMEM((128, 32), jnp.float32),
            pltpu.VMEM((128, 32), jnp.float32),
            pltpu.VMEM((128, 32), jnp.float32),
            pltpu.VMEM_SHARED((N_TOT, 32), jnp.float32),
            pltpu.SemaphoreType.DMA,
            pltpu.SemaphoreType.DMA,
            pltpu.SemaphoreType.DMA,
            pltpu.SemaphoreType.DMA,
            pltpu.SemaphoreType.DMA,
            pltpu.SemaphoreType.DMA,
            pltpu.SemaphoreType.DMA,
            pltpu.SemaphoreType.DMA,
        ],
        compiler_params=pltpu.CompilerParams(use_tc_tiling_on_sc=False),
    )
    return f(hsq[0], hsq[1], hsq[2], hsq[3], row2d, col2d, ew2d)


_R = 1000


def _k1_body(x_ref, w_ref, p0_ref, p1_ref, q0, q1, q2, q3):
    dinv = lax.rsqrt(1.0 + p0_ref[...] + p1_ref[...])
    hs = dinv * jnp.dot(x_ref[...], w_ref[...], preferred_element_type=jnp.float32)
    for i, o in enumerate((q0, q1, q2, q3)):
        o[...] = hs[:, 32 * i:32 * i + 32]


def _k1(x, w, p0c, p1c):
    full = lambda a: pl.BlockSpec(a.shape, lambda i: (0,) * a.ndim)
    return pl.pallas_call(
        _k1_body,
        grid=(N_TOT // _R,),
        in_specs=[pl.BlockSpec((_R, IN_CH), lambda i: (i, 0)), full(w),
                  pl.BlockSpec((_R, 1), lambda i: (i, 0)),
                  pl.BlockSpec((_R, 1), lambda i: (i, 0))],
        out_specs=[pl.BlockSpec((_R, 32), lambda i: (i, 0))] * 4,
        out_shape=[jax.ShapeDtypeStruct((N_TOT, 32), jnp.float32)] * 4,
    )(x, w, p0c, p1c)


def _k2_body(a0, a1, a2, a3, p0_ref, p1_ref, w_ref, b_ref, g_ref, be_ref,
             h1_ref, q0, q1, q2, q3):
    dinv = lax.rsqrt(1.0 + p0_ref[...] + p1_ref[...])
    agg = jnp.concatenate([a0[...], a1[...], a2[...], a3[...]], axis=1)
    h1 = g_ref[...] * (jax.nn.relu(dinv * agg + b_ref[...]) * BN_SCALE) + be_ref[...]
    h1_ref[...] = h1
    hs2 = dinv * jnp.dot(h1, w_ref[...], preferred_element_type=jnp.float32)
    for i, o in enumerate((q0, q1, q2, q3)):
        o[...] = hs2[:, 32 * i:32 * i + 32]


def _k2(aggq, p0c, p1c, w, b, g, be):
    full = lambda a: pl.BlockSpec(a.shape, lambda i: (0,) * a.ndim)
    return pl.pallas_call(
        _k2_body,
        grid=(N_TOT // _R,),
        in_specs=[pl.BlockSpec((_R, 32), lambda i: (i, 0))] * 4
        + [pl.BlockSpec((_R, 1), lambda i: (i, 0)),
           pl.BlockSpec((_R, 1), lambda i: (i, 0)),
           full(w), full(b), full(g), full(be)],
        out_specs=[pl.BlockSpec((_R, IN_CH), lambda i: (i, 0))]
        + [pl.BlockSpec((_R, 32), lambda i: (i, 0))] * 4,
        out_shape=[jax.ShapeDtypeStruct((N_TOT, IN_CH), jnp.float32)]
        + [jax.ShapeDtypeStruct((N_TOT, 32), jnp.float32)] * 4,
    )(aggq[0], aggq[1], aggq[2], aggq[3], p0c, p1c, w, b, g, be)


_B3 = 1000


def _k3_body(h1_ref, a0, a1, a2, a3, p0_ref, p1_ref, s_ref,
             bc2_ref, g2_ref, be2_ref,
             wih1_ref, whh1_ref, b1_ref, wih2_ref, whh2_ref, b2_ref,
             wl1a_ref, wl1b_ref, wl1c_ref, bl1_ref, wl2_ref, bl2_ref, out_ref):
    B = _B3
    h1s = jnp.zeros((B, HID), jnp.float32)
    c1s = jnp.zeros((B, HID), jnp.float32)
    h2s = jnp.zeros((B, HID), jnp.float32)
    c2s = jnp.zeros((B, HID), jnp.float32)
    wih1 = wih1_ref[...]
    whh1 = whh1_ref[...]
    wih2 = wih2_ref[...]
    whh2 = whh2_ref[...]
    b1 = b1_ref[...]
    b2 = b2_ref[...]

    def lstm_step(g, c):
        i = jax.nn.sigmoid(g[:, 0:HID])
        f = jax.nn.sigmoid(g[:, HID:2 * HID])
        gg = jnp.tanh(g[:, 2 * HID:3 * HID])
        o = jax.nn.sigmoid(g[:, 3 * HID:4 * HID])
        c = f * c + i * gg
        return o * jnp.tanh(c), c

    for t in range(WINDOW):
        dinv = lax.rsqrt(1.0 + p0_ref[t] + p1_ref[t])
        agg = jnp.concatenate([a0[t], a1[t], a2[t], a3[t]], axis=1)
        h2t = g2_ref[...] * (jax.nn.relu(dinv * agg + bc2_ref[...]) * BN_SCALE) + be2_ref[...]
        xt1 = h1_ref[t]
        gates1 = (jnp.dot(xt1, wih1[0:HID], preferred_element_type=jnp.float32)
                  + jnp.dot(h2t, wih1[HID:2 * HID], preferred_element_type=jnp.float32)
                  + jnp.dot(h1s, whh1, preferred_element_type=jnp.float32) + b1)
        h1s, c1s = lstm_step(gates1, c1s)
        gates2 = (jnp.dot(h1s, wih2, preferred_element_type=jnp.float32)
                  + jnp.dot(h2s, whh2, preferred_element_type=jnp.float32) + b2)
        h2s, c2s = lstm_step(gates2, c2s)

    ht = jnp.dot(jax.nn.relu(h1s), wl1a_ref[...], preferred_element_type=jnp.float32)
    ht = ht + jnp.dot(jax.nn.relu(h2s), wl1b_ref[...], preferred_element_type=jnp.float32)
    ht = ht + jnp.dot(jax.nn.relu(s_ref[...]), wl1c_ref[...], preferred_element_type=jnp.float32)
    ht = jax.nn.relu(ht + bl1_ref[...])
    out_ref[...] = jnp.dot(ht, wl2_ref[...], preferred_element_type=jnp.float32) + bl2_ref[...]


def _k3(h1r, aggr, p0r, p1r, s, bc2, g2, be2,
        wih1t, whh1t, b1, wih2t, whh2t, b2, wl1a, wl1b, wl1c, bl1, wl2t, bl2):
    full = lambda a: pl.BlockSpec(a.shape, lambda i: (0,) * a.ndim)
    B = _B3
    return pl.pallas_call(
        _k3_body,
        grid=(NUM_NODES // B,),
        in_specs=[pl.BlockSpec((WINDOW, B, IN_CH), lambda i: (0, i, 0))]
        + [pl.BlockSpec((WINDOW, B, 32), lambda i: (0, i, 0))] * 4
        + [pl.BlockSpec((WINDOW, B, 1), lambda i: (0, i, 0))] * 2
        + [pl.BlockSpec((B, s.shape[1]), lambda i: (i, 0))]
        + [full(a) for a in (bc2, g2, be2, wih1t, whh1t, b1, wih2t, whh2t, b2,
                             wl1a, wl1b, wl1c, bl1, wl2t, bl2)],
        out_specs=pl.BlockSpec((B, 7), lambda i: (i, 0)),
        out_shape=jax.ShapeDtypeStruct((NUM_NODES, 7), jnp.float32),
    )(h1r, aggr[0], aggr[1], aggr[2], aggr[3], p0r, p1r, s,
      bc2, g2, be2, wih1t, whh1t, b1, wih2t, whh2t, b2,
      wl1a, wl1b, wl1c, bl1, wl2t, bl2)


def kernel(x, edge_index, edge_weight, Wc1, bc1, Wc2, bc2, g1, be1, g2, be2,
           Wih1, Whh1, bih1, bhh1, Wih2, Whh2, bih2, bhh2, Wl1, bl1, Wl2, bl2):
    pad = E_PAD - N_EDGES
    pidx = (jnp.arange(pad, dtype=jnp.int32) * 79) % N_TOT
    row2d = jnp.concatenate([edge_index[0], pidx]).reshape(EROWS, 128)
    col2d = jnp.concatenate([edge_index[1], pidx]).reshape(EROWS, 128)
    ew2d = jnp.concatenate([edge_weight, jnp.zeros((pad,), jnp.float32)]).reshape(EROWS, 128)

    p0, p1 = _sc_deg(col2d, ew2d)
    p0c = p0.reshape(N_TOT, 1)
    p1c = p1.reshape(N_TOT, 1)

    hs1q = _k1(x, Wc1, p0c, p1c)
    agg1q = _sc_agg(hs1q, row2d, col2d, ew2d)
    h1, *hs2q = _k2(agg1q, p0c, p1c, Wc2, bc1.reshape(1, HID),
                    g1.reshape(1, HID), be1.reshape(1, HID))
    agg2q = _sc_agg(hs2q, row2d, col2d, ew2d)

    s = jnp.concatenate(
        [x[0:NUM_NODES]]
        + [x[t * NUM_NODES:(t + 1) * NUM_NODES, IN_CH - 1:IN_CH] for t in range(1, WINDOW)],
        axis=1)

    wl1t = Wl1.T
    return _k3(
        h1.reshape(WINDOW, NUM_NODES, IN_CH),
        [a.reshape(WINDOW, NUM_NODES, 32) for a in agg2q],
        p0.reshape(WINDOW, NUM_NODES, 1), p1.reshape(WINDOW, NUM_NODES, 1),
        s, bc2.reshape(1, HID), g2.reshape(1, HID), be2.reshape(1, HID),
        Wih1.T, Whh1.T, (bih1 + bhh1).reshape(1, 4 * HID),
        Wih2.T, Whh2.T, (bih2 + bhh2).reshape(1, 4 * HID),
        wl1t[0:HID], wl1t[HID:2 * HID], wl1t[2 * HID:],
        bl1.reshape(1, HID), Wl2.T, bl2.reshape(1, 7))
